# baseline (speedup 1.0000x reference)
import functools

import jax
import jax.numpy as jnp
from jax import lax
from jax.experimental import pallas as pl
from jax.experimental.pallas import tpu as pltpu
from jax.experimental.pallas import tpu_sc as plsc

N = 10000
E = 320000
D = 128
EPS = 1e-5

NC, NS, L = 2, 16, 16
NW = NC * NS
K = 128
CH = 80
E_PAD = NW * CH * K
PH = 40
NT = N + K
ZR = 624
ZTAIL = N - NS * ZR

_mesh = plsc.VectorSubcoreMesh(core_axis_name="c", subcore_axis_name="s")


@functools.partial(
    pl.kernel,
    out_type=(jax.ShapeDtypeStruct((NC, 1, NT), jnp.float32),
              jax.ShapeDtypeStruct((NC, 1, NT), jnp.float32)),
    mesh=_mesh,
    scratch_types=[
        pltpu.VMEM((CH, K), jnp.int32),
        pltpu.VMEM((CH, K), jnp.int32),
        pltpu.VMEM((K,), jnp.float32),
        pltpu.VMEM((NT,), jnp.float32),
        pltpu.VMEM_SHARED((NT,), jnp.float32),
        pltpu.VMEM_SHARED((NT,), jnp.float32),
    ],
)
def _sc_degrees(src_hbm, dst_hbm, dgo_hbm, dgi_hbm,
                src_v, dst_v, ones_v, zb_v, dgo, dgi):
    c = lax.axis_index("c")
    s = lax.axis_index("s")
    wid = s * NC + c
    pltpu.sync_copy(src_hbm.at[pl.ds(wid * CH, CH)], src_v)
    pltpu.sync_copy(dst_hbm.at[pl.ds(wid * CH, CH)], dst_v)
    for j in range(K // L):
        ones_v[pl.ds(j * L, L)] = jnp.ones((L,), jnp.float32)

    @pl.when(s == 0)
    def _zero():
        def zz(i, carry):
            zb_v[pl.ds(i * L, L)] = jnp.zeros((L,), jnp.float32)
            return carry
        lax.fori_loop(0, NT // L, zz, 0)
        pltpu.sync_copy(zb_v, dgo)
        pltpu.sync_copy(zb_v, dgi)

    plsc.subcore_barrier()

    def body(i, carry):
        pltpu.sync_copy(ones_v, dgo.at[src_v.at[i]], add=True)
        pltpu.sync_copy(ones_v, dgi.at[dst_v.at[i]], add=True)
        return carry
    lax.fori_loop(0, CH, body, 0)

    plsc.subcore_barrier()

    @pl.when(s == 0)
    def _writeback():
        pltpu.sync_copy(dgo, dgo_hbm.at[c, 0])
        pltpu.sync_copy(dgi, dgi_hbm.at[c, 0])


@functools.partial(
    pl.kernel,
    out_type=jax.ShapeDtypeStruct((NC, N, D), jnp.float32),
    mesh=_mesh,
    scratch_types=[
        pltpu.VMEM((PH, K), jnp.int32),
        pltpu.VMEM((PH, K), jnp.int32),
        pltpu.VMEM((K, D), jnp.float32),
        pltpu.VMEM((K, D), jnp.float32),
        pltpu.VMEM_SHARED((NT, D), jnp.float32),
        pltpu.SemaphoreType.DMA,
        pltpu.SemaphoreType.DMA,
        pltpu.SemaphoreType.DMA,
        pltpu.SemaphoreType.DMA,
    ],
)
def _sc_scatter(z_hbm, src_hbm, dst_hbm, out_hbm,
                src_v, dst_v, buf_a, buf_b, agg, sem_a, sem_b,
                sem_sa, sem_sb):
    c = lax.axis_index("c")
    s = lax.axis_index("s")
    wid = s * NC + c

    def zb(i, carry):
        for j in range(D // L):
            buf_a[i, pl.ds(j * L, L)] = jnp.zeros((L,), jnp.float32)
        return carry
    lax.fori_loop(0, K, zb, 0)
    for t in range(4):
        pltpu.sync_copy(buf_a.at[pl.ds(0, K)],
                        agg.at[pl.ds(s * ZR + t * K, K)])
    pltpu.sync_copy(buf_a.at[pl.ds(0, ZR - 4 * K)],
                    agg.at[pl.ds(s * ZR + 4 * K, ZR - 4 * K)])

    @pl.when(s == 0)
    def _zero_tail():
        pltpu.sync_copy(buf_a.at[pl.ds(0, ZTAIL)],
                        agg.at[pl.ds(NS * ZR, ZTAIL)])

    plsc.subcore_barrier()

    def fire(i, buf, sem):
        pltpu.async_copy(z_hbm.at[src_v.at[i]], buf, sem)

    def wait(i, buf, sem):
        pltpu.make_async_copy(z_hbm.at[src_v.at[i]], buf, sem).wait()

    def fire_s(i, buf, sem):
        pltpu.async_copy(buf, agg.at[dst_v.at[i]], sem, add=True)

    def wait_s(i, buf, sem):
        pltpu.make_async_copy(buf, agg.at[dst_v.at[i]], sem).wait()

    def run_phase(base, nch):
        pltpu.sync_copy(src_hbm.at[pl.ds(base, nch)], src_v.at[pl.ds(0, nch)])
        pltpu.sync_copy(dst_hbm.at[pl.ds(base, nch)], dst_v.at[pl.ds(0, nch)])

        fire(0, buf_a, sem_a)
        fire(1, buf_b, sem_b)

        def body(j, carry):
            i0 = 2 * j
            wait(i0, buf_a, sem_a)
            fire_s(i0, buf_a, sem_sa)
            wait(i0 + 1, buf_b, sem_b)
            fire_s(i0 + 1, buf_b, sem_sb)
            wait_s(i0, buf_a, sem_sa)
            fire(i0 + 2, buf_a, sem_a)
            wait_s(i0 + 1, buf_b, sem_sb)
            fire(i0 + 3, buf_b, sem_b)
            return carry
        lax.fori_loop(0, nch // 2 - 1, body, 0)

        wait(nch - 2, buf_a, sem_a)
        fire_s(nch - 2, buf_a, sem_sa)
        wait(nch - 1, buf_b, sem_b)
        fire_s(nch - 1, buf_b, sem_sb)
        wait_s(nch - 2, buf_a, sem_sa)
        wait_s(nch - 1, buf_b, sem_sb)

    for p in range(CH // PH):
        with jax.named_scope("edge_phase_%d" % p):
            run_phase(wid * CH + p * PH, PH)

    plsc.subcore_barrier()

    with jax.named_scope("agg_writeback"):
        pltpu.sync_copy(agg.at[pl.ds(s * ZR, ZR)],
                        out_hbm.at[c, pl.ds(s * ZR, ZR)])

        @pl.when(s == 0)
        def _write_tail():
            pltpu.sync_copy(agg.at[pl.ds(NS * ZR, ZTAIL)],
                            out_hbm.at[c, pl.ds(NS * ZR, ZTAIL)])


def _rsqrt(x):
    y = lax.rsqrt(x)
    return y * (1.5 - 0.5 * x * y * y)


def _norms(dgo_ref, dgi_ref):
    deg_out = dgo_ref[0, 0, :N] + dgo_ref[1, 0, :N]
    deg_in = dgi_ref[0, 0, :N] + dgi_ref[1, 0, :N]
    no = _rsqrt(jnp.maximum(deg_out, 1.0))
    ni = _rsqrt(jnp.maximum(deg_in, 1.0))
    return no, ni


def _bn_relu(t, g, b):
    mean = jnp.mean(t, axis=0)
    var = jnp.mean(jnp.square(t - mean[None, :]), axis=0)
    h = (t - mean[None, :]) * _rsqrt(var + EPS)[None, :] * g[None, :] + b[None, :]
    return jnp.maximum(h, 0.0)


def _tc1_body(dgo_ref, dgi_ref, x_ref, z_ref):
    no, _ = _norms(dgo_ref, dgi_ref)
    z_ref[:N, :] = x_ref[...] * no[:, None]
    z_ref[N:, :] = jnp.zeros((NT - N, D), jnp.float32)


_tc1 = pl.pallas_call(
    _tc1_body, out_shape=jax.ShapeDtypeStruct((NT, D), jnp.float32))


def _tc2_body(dgo_ref, dgi_ref, sp_ref, w_ref, g_ref, b_ref, z_ref):
    no, ni = _norms(dgo_ref, dgi_ref)
    t = (sp_ref[0] + sp_ref[1]) * ni[:, None]
    u = jnp.dot(t, w_ref[...], preferred_element_type=jnp.float32)
    h = _bn_relu(u, g_ref[...], b_ref[...])
    z_ref[:N, :] = h * no[:, None]
    z_ref[N:, :] = jnp.zeros((NT - N, D), jnp.float32)


_tc2 = pl.pallas_call(
    _tc2_body, out_shape=jax.ShapeDtypeStruct((NT, D), jnp.float32))


def _tc3_body(dgo_ref, dgi_ref, sp_ref, w_ref, g_ref, b_ref, o_ref):
    _, ni = _norms(dgo_ref, dgi_ref)
    t = (sp_ref[0] + sp_ref[1]) * ni[:, None]
    u = jnp.dot(t, w_ref[...], preferred_element_type=jnp.float32)
    o_ref[...] = _bn_relu(u, g_ref[...], b_ref[...])


_tc3 = pl.pallas_call(
    _tc3_body, out_shape=jax.ShapeDtypeStruct((N, D), jnp.float32))


def kernel(node_features, edge_index, W1, b1, gamma1, beta1,
           W2, b2, gamma2, beta2):
    src = edge_index[0].astype(jnp.int32)
    dst = edge_index[1].astype(jnp.int32)
    pad = N + (jnp.arange(E_PAD - E, dtype=jnp.int32) % K)
    src2 = jnp.concatenate([src, pad]).reshape(NW * CH, K)
    dst2 = jnp.concatenate([dst, pad]).reshape(NW * CH, K)

    dgo, dgi = _sc_degrees(src2, dst2)
    h1 = _tc1(dgo, dgi, node_features)
    s1 = _sc_scatter(h1, src2, dst2)
    h2 = _tc2(dgo, dgi, s1, W1, gamma1, beta1)
    s2 = _sc_scatter(h2, src2, dst2)
    return _tc3(dgo, dgi, s2, W2, gamma2, beta2)


# speedup vs baseline: 1.2402x; 1.2402x over previous
import functools

import jax
import jax.numpy as jnp
from jax import lax
from jax.experimental import pallas as pl
from jax.experimental.pallas import tpu as pltpu
from jax.experimental.pallas import tpu_sc as plsc

N = 10000
E = 320000
D = 128
EPS = 1e-5

NC, NS, L = 2, 16, 16
NW = NC * NS
K = 128
CH = 80
E_PAD = NW * CH * K
PH = 40
NT = N + K
ZR = 624
ZTAIL = N - NS * ZR

_mesh = plsc.VectorSubcoreMesh(core_axis_name="c", subcore_axis_name="s")


@functools.partial(
    pl.kernel,
    out_type=(jax.ShapeDtypeStruct((NC, 1, NT), jnp.float32),
              jax.ShapeDtypeStruct((NC, 1, NT), jnp.float32)),
    mesh=_mesh,
    scratch_types=[
        pltpu.VMEM((CH, K), jnp.int32),
        pltpu.VMEM((CH, K), jnp.int32),
        pltpu.VMEM((K,), jnp.float32),
        pltpu.VMEM((NT,), jnp.float32),
        pltpu.VMEM_SHARED((NT,), jnp.float32),
        pltpu.VMEM_SHARED((NT,), jnp.float32),
    ],
)
def _sc_degrees(src_hbm, dst_hbm, dgo_hbm, dgi_hbm,
                src_v, dst_v, ones_v, zb_v, dgo, dgi):
    c = lax.axis_index("c")
    s = lax.axis_index("s")
    wid = s * NC + c
    pltpu.sync_copy(src_hbm.at[pl.ds(wid * CH, CH)], src_v)
    pltpu.sync_copy(dst_hbm.at[pl.ds(wid * CH, CH)], dst_v)
    for j in range(K // L):
        ones_v[pl.ds(j * L, L)] = jnp.ones((L,), jnp.float32)

    @pl.when(s == 0)
    def _zero():
        def zz(i, carry):
            zb_v[pl.ds(i * L, L)] = jnp.zeros((L,), jnp.float32)
            return carry
        lax.fori_loop(0, NT // L, zz, 0)
        pltpu.sync_copy(zb_v, dgo)
        pltpu.sync_copy(zb_v, dgi)

    plsc.subcore_barrier()

    def body(i, carry):
        pltpu.sync_copy(ones_v, dgo.at[src_v.at[i]], add=True)
        pltpu.sync_copy(ones_v, dgi.at[dst_v.at[i]], add=True)
        return carry
    lax.fori_loop(0, CH, body, 0)

    plsc.subcore_barrier()

    @pl.when(s == 0)
    def _writeback():
        pltpu.sync_copy(dgo, dgo_hbm.at[c, 0])
        pltpu.sync_copy(dgi, dgi_hbm.at[c, 0])


@functools.partial(
    pl.kernel,
    out_type=jax.ShapeDtypeStruct((NC, N, D), jnp.float32),
    mesh=_mesh,
    scratch_types=[
        pltpu.VMEM((PH, K), jnp.int32),
        pltpu.VMEM((PH, K), jnp.int32),
        pltpu.VMEM((K, D), jnp.float32),
        pltpu.VMEM((K, D), jnp.float32),
        pltpu.VMEM_SHARED((NT, D), jnp.float32),
        pltpu.SemaphoreType.DMA,
        pltpu.SemaphoreType.DMA,
    ],
)
def _sc_scatter(z_hbm, src_hbm, dst_hbm, out_hbm,
                src_v, dst_v, buf_a, buf_b, agg, sem_a, sem_b):
    c = lax.axis_index("c")
    s = lax.axis_index("s")
    wid = s * NC + c

    def zb(i, carry):
        for j in range(D // L):
            buf_a[i, pl.ds(j * L, L)] = jnp.zeros((L,), jnp.float32)
        return carry
    lax.fori_loop(0, K, zb, 0)
    for t in range(4):
        pltpu.sync_copy(buf_a.at[pl.ds(0, K)],
                        agg.at[pl.ds(s * ZR + t * K, K)])
    pltpu.sync_copy(buf_a.at[pl.ds(0, ZR - 4 * K)],
                    agg.at[pl.ds(s * ZR + 4 * K, ZR - 4 * K)])

    @pl.when(s == 0)
    def _zero_tail():
        pltpu.sync_copy(buf_a.at[pl.ds(0, ZTAIL)],
                        agg.at[pl.ds(NS * ZR, ZTAIL)])

    plsc.subcore_barrier()

    def fire(i, buf, sem):
        pltpu.async_copy(z_hbm.at[src_v.at[i]], buf, sem)

    def wait(i, buf, sem):
        pltpu.make_async_copy(z_hbm.at[src_v.at[i]], buf, sem).wait()

    def scat(i, buf):
        pltpu.sync_copy(buf, agg.at[dst_v.at[i]], add=True)

    def run_phase(base, nch):
        pltpu.sync_copy(src_hbm.at[pl.ds(base, nch)], src_v.at[pl.ds(0, nch)])
        pltpu.sync_copy(dst_hbm.at[pl.ds(base, nch)], dst_v.at[pl.ds(0, nch)])

        fire(0, buf_a, sem_a)

        def body(j, carry):
            i0 = 2 * j
            fire(i0 + 1, buf_b, sem_b)
            wait(i0, buf_a, sem_a)
            scat(i0, buf_a)
            fire(i0 + 2, buf_a, sem_a)
            wait(i0 + 1, buf_b, sem_b)
            scat(i0 + 1, buf_b)
            return carry
        lax.fori_loop(0, nch // 2 - 1, body, 0)

        fire(nch - 1, buf_b, sem_b)
        wait(nch - 2, buf_a, sem_a)
        scat(nch - 2, buf_a)
        wait(nch - 1, buf_b, sem_b)
        scat(nch - 1, buf_b)

    for p in range(CH // PH):
        with jax.named_scope("edge_phase_%d" % p):
            run_phase(wid * CH + p * PH, PH)

    plsc.subcore_barrier()

    with jax.named_scope("agg_writeback"):
        pltpu.sync_copy(agg.at[pl.ds(s * ZR, ZR)],
                        out_hbm.at[c, pl.ds(s * ZR, ZR)])

        @pl.when(s == 0)
        def _write_tail():
            pltpu.sync_copy(agg.at[pl.ds(NS * ZR, ZTAIL)],
                            out_hbm.at[c, pl.ds(NS * ZR, ZTAIL)])


def _rsqrt(x):
    y = lax.rsqrt(x)
    return y * (1.5 - 0.5 * x * y * y)


def _norms(dgo_ref, dgi_ref):
    deg_out = dgo_ref[0, 0, :N] + dgo_ref[1, 0, :N]
    deg_in = dgi_ref[0, 0, :N] + dgi_ref[1, 0, :N]
    no = _rsqrt(jnp.maximum(deg_out, 1.0))
    ni = _rsqrt(jnp.maximum(deg_in, 1.0))
    return no, ni


def _bn_relu(t, g, b):
    mean = jnp.mean(t, axis=0)
    var = jnp.mean(jnp.square(t - mean[None, :]), axis=0)
    h = (t - mean[None, :]) * _rsqrt(var + EPS)[None, :] * g[None, :] + b[None, :]
    return jnp.maximum(h, 0.0)


def _tc1_body(dgo_ref, dgi_ref, x_ref, z_ref):
    no, _ = _norms(dgo_ref, dgi_ref)
    z_ref[:N, :] = x_ref[...] * no[:, None]
    z_ref[N:, :] = jnp.zeros((NT - N, D), jnp.float32)


_tc1 = pl.pallas_call(
    _tc1_body, out_shape=jax.ShapeDtypeStruct((NT, D), jnp.float32))


def _tc2_body(dgo_ref, dgi_ref, sp_ref, w_ref, g_ref, b_ref, z_ref):
    no, ni = _norms(dgo_ref, dgi_ref)
    t = (sp_ref[0] + sp_ref[1]) * ni[:, None]
    u = jnp.dot(t, w_ref[...], preferred_element_type=jnp.float32)
    h = _bn_relu(u, g_ref[...], b_ref[...])
    z_ref[:N, :] = h * no[:, None]
    z_ref[N:, :] = jnp.zeros((NT - N, D), jnp.float32)


_tc2 = pl.pallas_call(
    _tc2_body, out_shape=jax.ShapeDtypeStruct((NT, D), jnp.float32))


def _tc3_body(dgo_ref, dgi_ref, sp_ref, w_ref, g_ref, b_ref, o_ref):
    _, ni = _norms(dgo_ref, dgi_ref)
    t = (sp_ref[0] + sp_ref[1]) * ni[:, None]
    u = jnp.dot(t, w_ref[...], preferred_element_type=jnp.float32)
    o_ref[...] = _bn_relu(u, g_ref[...], b_ref[...])


_tc3 = pl.pallas_call(
    _tc3_body, out_shape=jax.ShapeDtypeStruct((N, D), jnp.float32))


def kernel(node_features, edge_index, W1, b1, gamma1, beta1,
           W2, b2, gamma2, beta2):
    src = edge_index[0].astype(jnp.int32)
    dst = edge_index[1].astype(jnp.int32)
    pad = N + (jnp.arange(E_PAD - E, dtype=jnp.int32) % K)
    src2 = jnp.concatenate([src, pad]).reshape(NW * CH, K)
    dst2 = jnp.concatenate([dst, pad]).reshape(NW * CH, K)

    dgo, dgi = _sc_degrees(src2, dst2)
    h1 = _tc1(dgo, dgi, node_features)
    s1 = _sc_scatter(h1, src2, dst2)
    h2 = _tc2(dgo, dgi, s1, W1, gamma1, beta1)
    s2 = _sc_scatter(h2, src2, dst2)
    return _tc3(dgo, dgi, s2, W2, gamma2, beta2)
